# baseline (speedup 1.0000x reference)
import functools

import jax
import jax.numpy as jnp
from jax import lax
from jax.experimental import pallas as pl
from jax.experimental.pallas import tpu as pltpu
from jax.experimental.pallas import tpu_sc as plsc

N = 10000
NPAD = 10240
E = 320000
EPAD = 327680
C = 32
NC = 2
NS = 16
NW = NC * NS
EPW = EPAD // NW
WIN = 128
NWINS = EPW // WIN
RPT = NPAD // NS
EPS = 1e-5

_MESH = plsc.VectorSubcoreMesh(
    core_axis_name="c", subcore_axis_name="s", num_cores=NC, num_subcores=NS
)


@functools.partial(
    pl.kernel,
    out_type=jax.ShapeDtypeStruct((NC, NPAD, C), jnp.float32),
    mesh=_MESH,
    compiler_params=pltpu.CompilerParams(use_tc_tiling_on_sc=False),
    scratch_types=(
        [pltpu.VMEM((NWINS, WIN), jnp.int32)] * 2
        + [pltpu.VMEM((WIN, C), jnp.float32)] * 8
        + [pltpu.VMEM((RPT, C), jnp.float32)]
        + [pltpu.VMEM_SHARED((NPAD, C), jnp.float32)]
        + [pltpu.SemaphoreType.DMA] * 16
    ),
)
def _edge_scatter(t_hbm, src_hbm, dst_hbm, out_hbm, *scr):
    src_a, dst_a = scr[0], scr[1]
    rows = scr[2:10]
    zbuf, acc_sh = scr[10], scr[11]
    t_sh = t_hbm
    gsem = scr[12:20]
    ssem = scr[20:28]
    B = 8

    cid = lax.axis_index("c")
    sid = lax.axis_index("s")
    wid = cid * NS + sid

    pltpu.sync_copy(src_hbm.at[pl.ds(wid * NWINS, NWINS)], src_a)
    pltpu.sync_copy(dst_hbm.at[pl.ds(wid * NWINS, NWINS)], dst_a)

    zeros = jnp.zeros((16,), jnp.float32)

    @pl.loop(0, RPT)
    def _zero(r):
        zbuf[r, pl.ds(0, 16)] = zeros
        zbuf[r, pl.ds(16, 16)] = zeros

    pltpu.sync_copy(zbuf, acc_sh.at[pl.ds(sid * RPT, RPT)])
    plsc.subcore_barrier()

    def g_start(w, b):
        pltpu.make_async_copy(t_sh.at[src_a.at[w]], rows[b], gsem[b]).start()

    def g_wait(w, b):
        pltpu.make_async_copy(t_sh.at[src_a.at[w]], rows[b], gsem[b]).wait()

    def s_start(w, b):
        pltpu.async_copy(rows[b], acc_sh.at[dst_a.at[w]], ssem[b], add=True)

    def s_wait(w, b):
        pltpu.make_async_copy(rows[b], acc_sh.at[dst_a.at[w]], ssem[b]).wait()

    for b in range(B):
        g_start(b, b)

    @pl.loop(0, NWINS - B, step=B)
    def _main(w0):
        for b in range(B):
            g_wait(w0 + b, b)
            s_start(w0 + b, b)
        for b in range(B):
            s_wait(w0 + b, b)
            g_start(w0 + B + b, b)

    for b in range(B):
        g_wait(NWINS - B + b, b)
        s_start(NWINS - B + b, b)
    for b in range(B):
        s_wait(NWINS - B + b, b)

    plsc.subcore_barrier()
    pltpu.sync_copy(acc_sh.at[pl.ds(sid * RPT, RPT)],
                    out_hbm.at[cid, pl.ds(sid * RPT, RPT)])


N4 = N // 4
NP4 = NPAD // 4


def _mm_body(x_ref, w_ref, o_ref):
    cols = [jnp.dot(x_ref[:, a, :], w_ref[...],
                    preferred_element_type=jnp.float32) for a in range(4)]
    o_ref[:N4] = jnp.concatenate(cols, axis=1)
    o_ref[N4:] = jnp.zeros((NP4 - N4, 128), jnp.float32)


def _fold4(v):
    return v[:, 0:32] + v[:, 32:64] + v[:, 64:96] + v[:, 96:128]


def _bn_scale_shift(hp, g_ref, b_ref):
    s1 = _fold4(jnp.sum(hp, axis=0, keepdims=True))
    s2 = _fold4(jnp.sum(hp * hp, axis=0, keepdims=True))
    mu = s1 * (1.0 / N)
    var = s2 * (1.0 / N) - mu * mu
    scale = g_ref[...] * lax.rsqrt(var + EPS)
    shift = b_ref[...] - mu * scale
    scale128 = jnp.concatenate([scale] * 4, axis=1)
    shift128 = jnp.concatenate([shift] * 4, axis=1)
    return scale128, shift128


def _bn_mm_body(p_ref, g_ref, b_ref, w_ref, t_ref, h_ref):
    hp = p_ref[0, :N4] + p_ref[1, :N4]
    scale, shift = _bn_scale_shift(hp, g_ref, b_ref)
    h = jnp.maximum(hp * scale + shift, 0.0)
    h_ref[...] = h
    t_ref[:N4] = jnp.dot(h, w_ref[...], preferred_element_type=jnp.float32)
    t_ref[N4:] = jnp.zeros((NP4 - N4, 128), jnp.float32)


def _final_body(p_ref, g_ref, b_ref, h2_ref, o_ref):
    hp = p_ref[0, :N4] + p_ref[1, :N4]
    scale, shift = _bn_scale_shift(hp, g_ref, b_ref)
    r2 = hp * scale + shift
    o_ref[...] = jnp.maximum(h2_ref[...] + r2, 0.0)


_f32 = jnp.float32
_mm = pl.pallas_call(_mm_body, out_shape=jax.ShapeDtypeStruct((NP4, 128), _f32))
_bn_mm = pl.pallas_call(
    _bn_mm_body,
    out_shape=(jax.ShapeDtypeStruct((NP4, 128), _f32),
               jax.ShapeDtypeStruct((N4, 128), _f32)),
)
_final = pl.pallas_call(_final_body,
                        out_shape=jax.ShapeDtypeStruct((N4, 128), _f32))


def kernel(x, edge_index, W0, g0, b0, W1, g1, b1, Wa, ga, ba, Wb, gb, bb):
    npad = EPAD - E
    pad_src = ((jnp.arange(npad, dtype=jnp.int32) * 131) % N).reshape(-1, WIN)
    pad_dst = (N + (jnp.arange(npad, dtype=jnp.int32) % (NPAD - N))).reshape(-1, WIN)
    ei3 = edge_index.reshape(2, E // WIN, WIN)
    src = jnp.concatenate([ei3[0], pad_src])
    dst = jnp.concatenate([ei3[1], pad_dst])

    g0, b0 = g0.reshape(1, C), b0.reshape(1, C)
    g1, b1 = g1.reshape(1, C), b1.reshape(1, C)
    ga, ba = ga.reshape(1, C), ba.reshape(1, C)
    gb, bb = gb.reshape(1, C), bb.reshape(1, C)

    eye4 = jnp.eye(4, dtype=_f32)
    W1b = jnp.kron(eye4, W1)
    Wab = jnp.kron(eye4, Wa)
    Wbb = jnp.kron(eye4, Wb)

    tp = _mm(x.reshape(N4, 4, 128), W0)
    p = _edge_scatter(tp.reshape(NPAD, C), src, dst)
    tp, _ = _bn_mm(p.reshape(NC, NP4, 128), g0, b0, W1b)
    p = _edge_scatter(tp.reshape(NPAD, C), src, dst)
    tp, h2 = _bn_mm(p.reshape(NC, NP4, 128), g1, b1, Wab)
    p = _edge_scatter(tp.reshape(NPAD, C), src, dst)
    tp, _ = _bn_mm(p.reshape(NC, NP4, 128), ga, ba, Wbb)
    p = _edge_scatter(tp.reshape(NPAD, C), src, dst)
    outp = _final(p.reshape(NC, NP4, 128), gb, bb, h2)
    return outp.reshape(N, C)


# speedup vs baseline: 1.0665x; 1.0665x over previous
import functools

import jax
import jax.numpy as jnp
from jax import lax
from jax.experimental import pallas as pl
from jax.experimental.pallas import tpu as pltpu
from jax.experimental.pallas import tpu_sc as plsc

N = 10000
NPAD = 10240
E = 320000
EPAD = 327680
C = 32
NC = 2
NS = 16
NW = NC * NS
EPW = EPAD // NW
WIN = 128
NWINS = EPW // WIN
RPT = NPAD // NS
EPS = 1e-5

_MESH = plsc.VectorSubcoreMesh(
    core_axis_name="c", subcore_axis_name="s", num_cores=NC, num_subcores=NS
)


@functools.partial(
    pl.kernel,
    out_type=jax.ShapeDtypeStruct((NC, NPAD, C), jnp.float32),
    mesh=_MESH,
    compiler_params=pltpu.CompilerParams(use_tc_tiling_on_sc=False),
    scratch_types=(
        [pltpu.VMEM((NWINS, WIN), jnp.int32)] * 2
        + [pltpu.VMEM((WIN, C), jnp.float32)] * 8
        + [pltpu.VMEM((RPT, C), jnp.float32)]
        + [pltpu.VMEM_SHARED((NPAD, C), jnp.float32)]
        + [pltpu.SemaphoreType.DMA] * 16
    ),
)
def _edge_scatter(t_hbm, src_hbm, dst_hbm, out_hbm, *scr):
    src_a, dst_a = scr[0], scr[1]
    rows = scr[2:10]
    zbuf, acc_sh = scr[10], scr[11]
    t_sh = t_hbm
    gsem = scr[12:20]
    ssem = scr[20:28]
    B = 8

    cid = lax.axis_index("c")
    sid = lax.axis_index("s")
    wid = cid * NS + sid

    idx_src = pltpu.make_async_copy(src_hbm.at[pl.ds(wid * NWINS, NWINS)],
                                    src_a, gsem[0])
    idx_dst = pltpu.make_async_copy(dst_hbm.at[pl.ds(wid * NWINS, NWINS)],
                                    dst_a, gsem[1])
    idx_src.start()
    idx_dst.start()

    zeros = jnp.zeros((16,), jnp.float32)

    @pl.loop(0, RPT)
    def _zero(r):
        zbuf[r, pl.ds(0, 16)] = zeros
        zbuf[r, pl.ds(16, 16)] = zeros

    pltpu.sync_copy(zbuf, acc_sh.at[pl.ds(sid * RPT, RPT)])
    idx_src.wait()
    idx_dst.wait()
    plsc.subcore_barrier()

    def g_start(w, b):
        pltpu.make_async_copy(t_sh.at[src_a.at[w]], rows[b], gsem[b]).start()

    def g_wait(w, b):
        pltpu.make_async_copy(t_sh.at[src_a.at[w]], rows[b], gsem[b]).wait()

    def s_start(w, b):
        pltpu.async_copy(rows[b], acc_sh.at[dst_a.at[w]], ssem[b], add=True)

    def s_wait(w, b):
        pltpu.make_async_copy(rows[b], acc_sh.at[dst_a.at[w]], ssem[b]).wait()

    for b in range(B):
        g_start(b, b)

    @pl.loop(0, NWINS - B, step=B)
    def _main(w0):
        for b in range(B):
            g_wait(w0 + b, b)
            s_start(w0 + b, b)
        for b in range(B):
            s_wait(w0 + b, b)
            g_start(w0 + B + b, b)

    for b in range(B):
        g_wait(NWINS - B + b, b)
        s_start(NWINS - B + b, b)
    for b in range(B):
        s_wait(NWINS - B + b, b)

    plsc.subcore_barrier()
    pltpu.sync_copy(acc_sh.at[pl.ds(sid * RPT, RPT)],
                    out_hbm.at[cid, pl.ds(sid * RPT, RPT)])


N4 = N // 4
NP4 = NPAD // 4


def _mm_body(x_ref, w_ref, o_ref):
    cols = [jnp.dot(x_ref[:, a, :], w_ref[...],
                    preferred_element_type=jnp.float32) for a in range(4)]
    o_ref[:N4] = jnp.concatenate(cols, axis=1)
    o_ref[N4:] = jnp.zeros((NP4 - N4, 128), jnp.float32)


def _fold4(v):
    return v[:, 0:32] + v[:, 32:64] + v[:, 64:96] + v[:, 96:128]


def _bn_scale_shift(hp, g_ref, b_ref):
    s1 = _fold4(jnp.sum(hp, axis=0, keepdims=True))
    s2 = _fold4(jnp.sum(hp * hp, axis=0, keepdims=True))
    mu = s1 * (1.0 / N)
    var = s2 * (1.0 / N) - mu * mu
    scale = g_ref[...] * lax.rsqrt(var + EPS)
    shift = b_ref[...] - mu * scale
    scale128 = jnp.concatenate([scale] * 4, axis=1)
    shift128 = jnp.concatenate([shift] * 4, axis=1)
    return scale128, shift128


def _bn_mm_body(p_ref, g_ref, b_ref, w_ref, t_ref, h_ref):
    hp = p_ref[0, :N4] + p_ref[1, :N4]
    scale, shift = _bn_scale_shift(hp, g_ref, b_ref)
    h = jnp.maximum(hp * scale + shift, 0.0)
    h_ref[...] = h
    t_ref[:N4] = jnp.dot(h, w_ref[...], preferred_element_type=jnp.float32)
    t_ref[N4:] = jnp.zeros((NP4 - N4, 128), jnp.float32)


def _final_body(p_ref, g_ref, b_ref, h2_ref, o_ref):
    hp = p_ref[0, :N4] + p_ref[1, :N4]
    scale, shift = _bn_scale_shift(hp, g_ref, b_ref)
    r2 = hp * scale + shift
    o_ref[...] = jnp.maximum(h2_ref[...] + r2, 0.0)


_f32 = jnp.float32
_mm = pl.pallas_call(_mm_body, out_shape=jax.ShapeDtypeStruct((NP4, 128), _f32))
_bn_mm = pl.pallas_call(
    _bn_mm_body,
    out_shape=(jax.ShapeDtypeStruct((NP4, 128), _f32),
               jax.ShapeDtypeStruct((N4, 128), _f32)),
)
_final = pl.pallas_call(_final_body,
                        out_shape=jax.ShapeDtypeStruct((N4, 128), _f32))


def kernel(x, edge_index, W0, g0, b0, W1, g1, b1, Wa, ga, ba, Wb, gb, bb):
    npad = EPAD - E
    pad_src = (jnp.arange(npad, dtype=jnp.int32) * 131) % N
    pad_dst = N + (jnp.arange(npad, dtype=jnp.int32) % (NPAD - N))
    src = jnp.concatenate([edge_index[0], pad_src]).reshape(EPAD // WIN, WIN)
    dst = jnp.concatenate([edge_index[1], pad_dst]).reshape(EPAD // WIN, WIN)

    g0, b0 = g0.reshape(1, C), b0.reshape(1, C)
    g1, b1 = g1.reshape(1, C), b1.reshape(1, C)
    ga, ba = ga.reshape(1, C), ba.reshape(1, C)
    gb, bb = gb.reshape(1, C), bb.reshape(1, C)

    eye4 = jnp.eye(4, dtype=_f32)
    W1b = jnp.kron(eye4, W1)
    Wab = jnp.kron(eye4, Wa)
    Wbb = jnp.kron(eye4, Wb)

    tp = _mm(x.reshape(N4, 4, 128), W0)
    p = _edge_scatter(tp.reshape(NPAD, C), src, dst)
    tp, _ = _bn_mm(p.reshape(NC, NP4, 128), g0, b0, W1b)
    p = _edge_scatter(tp.reshape(NPAD, C), src, dst)
    tp, h2 = _bn_mm(p.reshape(NC, NP4, 128), g1, b1, Wab)
    p = _edge_scatter(tp.reshape(NPAD, C), src, dst)
    tp, _ = _bn_mm(p.reshape(NC, NP4, 128), ga, ba, Wbb)
    p = _edge_scatter(tp.reshape(NPAD, C), src, dst)
    outp = _final(p.reshape(NC, NP4, 128), gb, bb, h2)
    return outp.reshape(N, C)
